# Initial kernel scaffold; baseline (speedup 1.0000x reference)
import jax
import jax.numpy as jnp
from jax.experimental import pallas as pl


def kernel(state_indices, state_table, subunit_table, W1, b1, W2, b2):
    raise NotImplementedError("write your pallas kernel here")


# speedup vs baseline: 5.8352x; 5.8352x over previous
import functools
import math

import jax
import jax.numpy as jnp
from jax import lax
from jax.experimental import pallas as pl
from jax.experimental.pallas import tpu as pltpu
from jax.experimental.pallas import tpu_sc as plsc

_EMBED = 64
_HALF = 32
_BATCH = 16384
_NSTATES = 3
_LUT_ROWS = 32

_NC = 2
_NS = 16
_NW = _NC * _NS
_BPW = _BATCH // _NW


def _lut_body(st_ref, sub_ref, w1_ref, b1_ref, w2_ref, b2_ref, lut_ref):
    t = lax.broadcasted_iota(jnp.int32, (_LUT_ROWS, 1), 0)
    a = t // 9
    b = (t // 3) % 3
    c = t % 3

    st = st_ref[...]
    sub = sub_ref[...]

    pooled_state = jnp.zeros((_LUT_ROWS, _HALF), jnp.float32)
    for k in range(_NSTATES):
        cnt = ((a == k).astype(jnp.float32)
               + (b == k).astype(jnp.float32)
               + (c == k).astype(jnp.float32))
        pooled_state = pooled_state + cnt * st[k : k + 1, :]
    pooled_state = pooled_state / 3.0

    sub_mean = (sub[0:1, :] + sub[1:2, :] + sub[2:3, :]) / 3.0
    pooled = jnp.concatenate(
        [pooled_state, jnp.broadcast_to(sub_mean, (_LUT_ROWS, _HALF))], axis=1
    )

    mean_idx = (a + b + c).astype(jnp.float32) / 3.0
    phase = (2.0 * math.pi / _NSTATES) * mean_idx
    p = lax.broadcasted_iota(jnp.int32, (1, _EMBED), 1)
    even_base = (p - p % 2).astype(jnp.float32)
    freq = jnp.exp(even_base * (-(math.log(10000.0) / _EMBED)))
    angles = phase * freq
    pe = jnp.where(p % 2 == 0, jnp.sin(angles), jnp.cos(angles))

    x = jnp.concatenate([pooled, pe], axis=1)

    h = jnp.dot(x, w1_ref[...], preferred_element_type=jnp.float32) + b1_ref[...]
    h = 0.5 * h * (1.0 + lax.erf(h / math.sqrt(2.0)))
    out = jnp.dot(h, w2_ref[...], preferred_element_type=jnp.float32) + b2_ref[...]

    norm = jnp.sqrt(jnp.sum(out * out, axis=1, keepdims=True))
    factor = jnp.minimum(jnp.ones_like(norm), 0.95 / (norm + 1e-8))
    lut_ref[...] = out * factor


def _sc_gather_body(lut_hbm, idx_hbm, out_hbm, tri_v, keys_v, rows_v, sem):
    wid = lax.axis_index("s") * _NC + lax.axis_index("c")
    base = wid * _BPW

    for col in range(3):
        pltpu.sync_copy(
            idx_hbm.at[pl.ds(col * _BATCH + base, _BPW)],
            tri_v.at[pl.ds(col * _BPW, _BPW)],
        )

    def grp(j, carry):
        lo = j * 16
        g0 = tri_v[pl.ds(lo, 16)]
        g1 = tri_v[pl.ds(_BPW + lo, 16)]
        g2 = tri_v[pl.ds(2 * _BPW + lo, 16)]
        keys_v[pl.ds(lo, 16)] = g0 * 9 + g1 * 3 + g2
        return carry

    lax.fori_loop(0, _BPW // 16, grp, 0)

    pltpu.async_copy(lut_hbm.at[keys_v], rows_v, sem).wait()
    pltpu.sync_copy(rows_v, out_hbm.at[pl.ds(base, _BPW)])


@functools.lru_cache(maxsize=1)
def _make_sc_gather():
    return pl.kernel(
        _sc_gather_body,
        out_type=jax.ShapeDtypeStruct((_BATCH, _EMBED), jnp.float32),
        scratch_types=[
            pltpu.VMEM((_BPW * 3,), jnp.int32),
            pltpu.VMEM((_BPW,), jnp.int32),
            pltpu.VMEM((_BPW, _EMBED), jnp.float32),
            pltpu.SemaphoreType.DMA,
        ],
        mesh=plsc.VectorSubcoreMesh(core_axis_name="c", subcore_axis_name="s"),
        compiler_params=pltpu.CompilerParams(use_tc_tiling_on_sc=False),
    )


def kernel(state_indices, state_table, subunit_table, W1, b1, W2, b2):
    lut = pl.pallas_call(
        _lut_body,
        out_shape=jax.ShapeDtypeStruct((_LUT_ROWS, _EMBED), jnp.float32),
    )(state_table, subunit_table, W1, b1, W2, b2)
    flat_idx = state_indices.T.reshape(-1).astype(jnp.int32)
    return _make_sc_gather()(lut, flat_idx)
